# baseline (speedup 1.0000x reference)
import functools

import jax
import jax.numpy as jnp
from jax.experimental import pallas as pl
from jax.experimental.pallas import tpu as pltpu

NUM_CLASSES = 100000
BLOCK_CLS = 1000
NBUF = 8


def _onehot_body(nsteps, b, users_ref, out_hbm, *bufs_and_sems):
    bufs = bufs_and_sems[:NBUF]
    sems = bufs_and_sems[NBUF:]
    j = pl.program_id(0)
    slot = jax.lax.rem(j, NBUF)

    rows = jax.lax.broadcasted_iota(jnp.int32, (BLOCK_CLS, b), 0) + j * BLOCK_CLS
    val = (users_ref[:, :] == rows).astype(jnp.float32)

    def _wait_prev(k):
        pltpu.make_async_copy(
            bufs[k],
            out_hbm.at[pl.ds((j - NBUF) * BLOCK_CLS, BLOCK_CLS), :],
            sems[k],
        ).wait()

    def _fill_and_send(k):
        bufs[k][...] = val
        pltpu.make_async_copy(
            bufs[k],
            out_hbm.at[pl.ds(j * BLOCK_CLS, BLOCK_CLS), :],
            sems[k],
        ).start(priority=k % 2)

    for k in range(NBUF):
        pl.when(jnp.logical_and(slot == k, j >= NBUF))(
            functools.partial(_wait_prev, k)
        )
        pl.when(slot == k)(functools.partial(_fill_and_send, k))

    @pl.when(j == nsteps - 1)
    def _drain():
        for step in range(max(0, nsteps - NBUF), nsteps):
            pltpu.make_async_copy(
                bufs[step % NBUF],
                out_hbm.at[pl.ds(step * BLOCK_CLS, BLOCK_CLS), :],
                sems[step % NBUF],
            ).wait()


def kernel(users):
    b = users.shape[0]
    nsteps = NUM_CLASSES // BLOCK_CLS
    users2 = users.reshape(1, b)
    scratch = [pltpu.VMEM((BLOCK_CLS, b), jnp.float32)] * NBUF
    dsems = [pltpu.SemaphoreType.DMA] * NBUF
    out_t = pl.pallas_call(
        functools.partial(_onehot_body, nsteps, b),
        grid=(nsteps,),
        in_specs=[pl.BlockSpec(memory_space=pltpu.MemorySpace.VMEM)],
        out_specs=pl.BlockSpec(memory_space=pltpu.MemorySpace.HBM),
        out_shape=jax.ShapeDtypeStruct((NUM_CLASSES, b), jnp.float32),
        scratch_shapes=scratch + dsems,
        compiler_params=pltpu.CompilerParams(
            vmem_limit_bytes=100 * 1024 * 1024,
        ),
    )(users2)
    return out_t.T


# speedup vs baseline: 1.0172x; 1.0172x over previous
import jax
import jax.numpy as jnp
from jax.experimental import pallas as pl

NUM_CLASSES = 100000
BLOCK_CLS = 1024


def _onehot_block(users_ref, out_ref):
    j = pl.program_id(0)
    rows = jax.lax.broadcasted_iota(jnp.int32, out_ref.shape, 0) + j * BLOCK_CLS
    out_ref[:, :] = (users_ref[:, :] == rows).astype(jnp.float32)


def kernel(users):
    b = users.shape[0]
    users2 = users.reshape(1, b)
    out_t = pl.pallas_call(
        _onehot_block,
        grid=(pl.cdiv(NUM_CLASSES, BLOCK_CLS),),
        in_specs=[pl.BlockSpec((1, b), lambda j: (0, 0))],
        out_specs=pl.BlockSpec((BLOCK_CLS, b), lambda j: (j, 0)),
        out_shape=jax.ShapeDtypeStruct((NUM_CLASSES, b), jnp.float32),
    )(users2)
    return out_t.T


# speedup vs baseline: 1.0369x; 1.0194x over previous
import jax
import jax.numpy as jnp
from jax.experimental import pallas as pl

NUM_CLASSES = 100000
BLOCK_CLS = 1000


def _onehot_block(users_ref, out_ref):
    j = pl.program_id(0)
    rows = jax.lax.broadcasted_iota(jnp.int32, out_ref.shape, 0) + j * BLOCK_CLS
    out_ref[:, :] = (users_ref[:, :] == rows).astype(jnp.float32)


def kernel(users):
    b = users.shape[0]
    users2 = users.reshape(1, b)
    out_t = pl.pallas_call(
        _onehot_block,
        grid=(pl.cdiv(NUM_CLASSES, BLOCK_CLS),),
        in_specs=[pl.BlockSpec((1, b), lambda j: (0, 0))],
        out_specs=pl.BlockSpec((BLOCK_CLS, b), lambda j: (j, 0)),
        out_shape=jax.ShapeDtypeStruct((NUM_CLASSES, b), jnp.float32),
    )(users2)
    return out_t.T
